# baseline (speedup 1.0000x reference)
import jax
import jax.numpy as jnp
from jax import lax
from jax.experimental import pallas as pl
from jax.experimental.pallas import tpu as pltpu, tpu_sc as plsc

NC, NS, L = 2, 16, 16
NW = NC * NS
B, A, C = 32, 65536, 8
KT = A // 128

KTC = 512
NG = B // 8
KB = 16
KPW = KT - KTC
CK = 48
NCHUNK = KPW // CK
ROWS = CK * C


def _sc_body(lp_hbm, lt_hbm, ct_hbm, out_hbm, lp_buf, lt_buf, ct_buf, res_buf):
    w = lax.axis_index("s") * NC + lax.axis_index("c")
    b = w
    r = b // 8
    i = b % 8
    kbase = KTC

    def chunk_body(c0, carry):
        k0 = kbase + c0 * CK
        pltpu.sync_copy(lp_hbm.at[b, pl.ds(k0 * C, ROWS), :], lp_buf)
        pltpu.sync_copy(lt_hbm.at[b, pl.ds(k0 * C, ROWS), :], lt_buf)
        pltpu.sync_copy(ct_hbm.at[r, pl.ds(k0, CK), i, :], ct_buf)

        def tile_body(kk, carry):
            accq, acca, acct, cnt = carry
            m = []
            for l in range(8):
                tl = ct_buf[kk, pl.ds(l * L, L)]
                ml = jnp.where(tl > 0, 1.0, 0.0).astype(jnp.float32)
                cnt = cnt + ml
                m.append(ml)
            for c in range(8):
                row = kk * 8 + c
                for l in range(8):
                    a = lp_buf[row, pl.ds(l * L, L)]
                    bb = lt_buf[row, pl.ds(l * L, L)]
                    d = (a - bb) * m[l]
                    absd = jnp.abs(d)
                    t = jnp.minimum(absd, 1.0)
                    accq = accq + (0.5 * t) * t
                    acca = acca + absd
                    acct = acct + t
            return accq, acca, acct, cnt

        return lax.fori_loop(0, CK, tile_body, carry)

    z = jnp.zeros((L,), jnp.float32)
    accq, acca, acct, cnt = lax.fori_loop(0, NCHUNK, chunk_body, (z, z, z, z))
    res_buf[pl.ds(0, L)] = accq + acca - acct
    res_buf[pl.ds(L, L)] = cnt
    pltpu.sync_copy(res_buf, out_hbm.at[w])


def _tc_body(lp_ref, lt_ref, ct_ref, sl1_out, cnt_out):
    kidx = pl.program_id(1)

    @pl.when(kidx == 0)
    def _():
        sl1_out[...] = jnp.zeros_like(sl1_out)
        cnt_out[...] = jnp.zeros_like(cnt_out)

    acc = jnp.zeros((8, 128), jnp.float32)
    cnt = jnp.zeros((8, 128), jnp.float32)
    for kk in range(KB):
        tl = ct_ref[:, kk * 128:(kk + 1) * 128]
        mask = tl > 0
        cnt = cnt + jnp.where(mask, 1.0, 0.0).astype(jnp.float32)
        for c in range(8):
            a = lp_ref[:, kk * 8 + c, :]
            bb = lt_ref[:, kk * 8 + c, :]
            d = jnp.where(mask, a - bb, 0.0)
            absd = jnp.abs(d)
            acc = acc + jnp.where(absd < 1.0, (0.5 * d) * d, absd - 0.5)
    sl1_out[0] += acc
    cnt_out[0] += cnt


def kernel(loc_preds, loc_targets, cls_preds, cls_targets):
    lp = loc_preds.reshape(B, KT, 128, C).transpose(0, 1, 3, 2).reshape(B, KT * C, 128)
    lt = loc_targets.reshape(B, KT, 128, C).transpose(0, 1, 3, 2).reshape(B, KT * C, 128)
    ct_sc = cls_targets.astype(jnp.int32).reshape(B // 8, 8, KT, 128).transpose(0, 2, 1, 3)
    ct_tc = cls_targets.astype(jnp.int32)

    mesh = plsc.VectorSubcoreMesh(
        core_axis_name="c", subcore_axis_name="s",
        num_cores=NC, num_subcores=NS)
    out_sc = pl.kernel(
        _sc_body,
        out_type=jax.ShapeDtypeStruct((NW, 2 * L), jnp.float32),
        mesh=mesh,
        scratch_types=[
            pltpu.VMEM((ROWS, 128), jnp.float32),
            pltpu.VMEM((ROWS, 128), jnp.float32),
            pltpu.VMEM((CK, 128), jnp.int32),
            pltpu.VMEM((2 * L,), jnp.float32),
        ],
    )(lp, lt, ct_sc)

    sl1_tc, cnt_tc = pl.pallas_call(
        _tc_body,
        grid=(NG, KTC // KB),
        in_specs=[
            pl.BlockSpec((8, KB * C, 128), lambda g, k: (g, k, 0)),
            pl.BlockSpec((8, KB * C, 128), lambda g, k: (g, k, 0)),
            pl.BlockSpec((8, KB * 128), lambda g, k: (g, k)),
        ],
        out_specs=[
            pl.BlockSpec((1, 8, 128), lambda g, k: (g, 0, 0)),
            pl.BlockSpec((1, 8, 128), lambda g, k: (g, 0, 0)),
        ],
        out_shape=[
            jax.ShapeDtypeStruct((NG, 8, 128), jnp.float32),
            jax.ShapeDtypeStruct((NG, 8, 128), jnp.float32),
        ],
        compiler_params=pltpu.CompilerParams(
            dimension_semantics=("parallel", "arbitrary")),
    )(lp, lt, ct_tc)

    sl1_sum = jnp.sum(out_sc[:, :L]) + jnp.sum(sl1_tc)
    n = jnp.sum(out_sc[:, L:]) + jnp.sum(cnt_tc)
    return 0.2 * (sl1_sum / n)


# speedup vs baseline: 1.2423x; 1.2423x over previous
import jax
import jax.numpy as jnp
from jax import lax
from jax.experimental import pallas as pl
from jax.experimental.pallas import tpu as pltpu, tpu_sc as plsc

NC, NS, L = 2, 16, 16
NW = NC * NS
B, A, C = 32, 65536, 8
KT = A // 128

KTC = 336
NG = B // 8
KB = 16
KPW = KT - KTC
CK = 44
NCHUNK = KPW // CK
ROWS = CK * C


def _sc_body(lp_hbm, lt_hbm, ct_hbm, out_hbm, lp_buf, lt_buf, ct_buf, res_buf):
    w = lax.axis_index("s") * NC + lax.axis_index("c")
    b = w
    r = b // 8
    i = b % 8
    kbase = KTC

    def chunk_body(c0, carry):
        k0 = kbase + c0 * CK
        pltpu.sync_copy(lp_hbm.at[b, pl.ds(k0 * C, ROWS), :], lp_buf)
        pltpu.sync_copy(lt_hbm.at[b, pl.ds(k0 * C, ROWS), :], lt_buf)
        pltpu.sync_copy(ct_hbm.at[r, pl.ds(k0, CK), i, :], ct_buf)

        def tile_body(kk, carry):
            accq, acca, acct, cnt = carry
            m = []
            for l in range(8):
                tl = ct_buf[kk, pl.ds(l * L, L)]
                ml = jnp.where(tl > 0, 1.0, 0.0).astype(jnp.float32)
                cnt = cnt + ml
                m.append(ml)
            for c in range(8):
                row = kk * 8 + c
                for l in range(8):
                    a = lp_buf[row, pl.ds(l * L, L)]
                    bb = lt_buf[row, pl.ds(l * L, L)]
                    d = (a - bb) * m[l]
                    absd = jnp.abs(d)
                    t = jnp.minimum(absd, 1.0)
                    accq = accq + (0.5 * t) * t
                    acca = acca + absd
                    acct = acct + t
            return accq, acca, acct, cnt

        return lax.fori_loop(0, CK, tile_body, carry)

    z = jnp.zeros((L,), jnp.float32)
    accq, acca, acct, cnt = lax.fori_loop(0, NCHUNK, chunk_body, (z, z, z, z))
    res_buf[pl.ds(0, L)] = accq + acca - acct
    res_buf[pl.ds(L, L)] = cnt
    pltpu.sync_copy(res_buf, out_hbm.at[w])


def _tc_body(lp_ref, lt_ref, ct_ref, sl1_out, cnt_out):
    kidx = pl.program_id(1)

    @pl.when(kidx == 0)
    def _():
        sl1_out[...] = jnp.zeros_like(sl1_out)
        cnt_out[...] = jnp.zeros_like(cnt_out)

    acc = jnp.zeros((8, 128), jnp.float32)
    cnt = jnp.zeros((8, 128), jnp.float32)
    for kk in range(KB):
        tl = ct_ref[:, kk * 128:(kk + 1) * 128]
        mask = tl > 0
        cnt = cnt + jnp.where(mask, 1.0, 0.0).astype(jnp.float32)
        for c in range(8):
            a = lp_ref[:, kk * 8 + c, :]
            bb = lt_ref[:, kk * 8 + c, :]
            d = jnp.where(mask, a - bb, 0.0)
            absd = jnp.abs(d)
            acc = acc + jnp.where(absd < 1.0, (0.5 * d) * d, absd - 0.5)
    sl1_out[0] += acc
    cnt_out[0] += cnt


def kernel(loc_preds, loc_targets, cls_preds, cls_targets):
    lp = loc_preds.reshape(B, KT, 128, C).transpose(0, 1, 3, 2).reshape(B, KT * C, 128)
    lt = loc_targets.reshape(B, KT, 128, C).transpose(0, 1, 3, 2).reshape(B, KT * C, 128)
    ct_sc = cls_targets.astype(jnp.int32).reshape(B // 8, 8, KT, 128).transpose(0, 2, 1, 3)
    ct_tc = cls_targets.astype(jnp.int32)

    mesh = plsc.VectorSubcoreMesh(
        core_axis_name="c", subcore_axis_name="s",
        num_cores=NC, num_subcores=NS)
    out_sc = pl.kernel(
        _sc_body,
        out_type=jax.ShapeDtypeStruct((NW, 2 * L), jnp.float32),
        mesh=mesh,
        scratch_types=[
            pltpu.VMEM((ROWS, 128), jnp.float32),
            pltpu.VMEM((ROWS, 128), jnp.float32),
            pltpu.VMEM((CK, 128), jnp.int32),
            pltpu.VMEM((2 * L,), jnp.float32),
        ],
    )(lp, lt, ct_sc)

    sl1_tc, cnt_tc = pl.pallas_call(
        _tc_body,
        grid=(NG, KTC // KB),
        in_specs=[
            pl.BlockSpec((8, KB * C, 128), lambda g, k: (g, k, 0)),
            pl.BlockSpec((8, KB * C, 128), lambda g, k: (g, k, 0)),
            pl.BlockSpec((8, KB * 128), lambda g, k: (g, k)),
        ],
        out_specs=[
            pl.BlockSpec((1, 8, 128), lambda g, k: (g, 0, 0)),
            pl.BlockSpec((1, 8, 128), lambda g, k: (g, 0, 0)),
        ],
        out_shape=[
            jax.ShapeDtypeStruct((NG, 8, 128), jnp.float32),
            jax.ShapeDtypeStruct((NG, 8, 128), jnp.float32),
        ],
        compiler_params=pltpu.CompilerParams(
            dimension_semantics=("parallel", "arbitrary")),
    )(lp, lt, ct_tc)

    sl1_sum = jnp.sum(out_sc[:, :L]) + jnp.sum(sl1_tc)
    n = jnp.sum(out_sc[:, L:]) + jnp.sum(cnt_tc)
    return 0.2 * (sl1_sum / n)


# speedup vs baseline: 1.3294x; 1.0701x over previous
import jax
import jax.numpy as jnp
from jax import lax
from jax.experimental import pallas as pl
from jax.experimental.pallas import tpu as pltpu, tpu_sc as plsc

NC, NS, L = 2, 16, 16
NW = NC * NS
B, A, C = 32, 65536, 8
KT = A // 128

KTC = 352
NG = B // 8
KB = 32
KPW = KT - KTC
CK = 40
NCHUNK = KPW // CK
ROWS = CK * C


def _sc_body(lp_hbm, lt_hbm, ct_hbm, out_hbm, lp_buf, lt_buf, ct_buf, res_buf):
    w = lax.axis_index("s") * NC + lax.axis_index("c")
    b = w
    r = b // 8
    i = b % 8
    kbase = KTC

    def chunk_body(c0, carry):
        k0 = kbase + c0 * CK
        pltpu.sync_copy(lp_hbm.at[b, pl.ds(k0 * C, ROWS), :], lp_buf)
        pltpu.sync_copy(lt_hbm.at[b, pl.ds(k0 * C, ROWS), :], lt_buf)
        pltpu.sync_copy(ct_hbm.at[r, pl.ds(k0, CK), i, :], ct_buf)

        def tile_body(kk, carry):
            accq, acca, acct, cnt = carry
            m = []
            for l in range(8):
                tl = ct_buf[kk, pl.ds(l * L, L)]
                ml = jnp.where(tl > 0, 1.0, 0.0).astype(jnp.float32)
                cnt = cnt + ml
                m.append(ml)
            for c in range(8):
                row = kk * 8 + c
                for l in range(8):
                    a = lp_buf[row, pl.ds(l * L, L)]
                    bb = lt_buf[row, pl.ds(l * L, L)]
                    d = (a - bb) * m[l]
                    absd = jnp.abs(d)
                    t = jnp.minimum(absd, 1.0)
                    accq = accq + (0.5 * t) * t
                    acca = acca + absd
                    acct = acct + t
            return accq, acca, acct, cnt

        return lax.fori_loop(0, CK, tile_body, carry)

    z = jnp.zeros((L,), jnp.float32)
    accq, acca, acct, cnt = lax.fori_loop(0, NCHUNK, chunk_body, (z, z, z, z))
    res_buf[pl.ds(0, L)] = accq + acca - acct
    res_buf[pl.ds(L, L)] = cnt
    pltpu.sync_copy(res_buf, out_hbm.at[w])


def _tc_body(lp_ref, lt_ref, ct_ref, sl1_out, cnt_out):
    kidx = pl.program_id(1)

    @pl.when(kidx == 0)
    def _():
        sl1_out[...] = jnp.zeros_like(sl1_out)
        cnt_out[...] = jnp.zeros_like(cnt_out)

    acc = jnp.zeros((8, 128), jnp.float32)
    cnt = jnp.zeros((8, 128), jnp.float32)
    for kk in range(KB):
        tl = ct_ref[:, kk * 128:(kk + 1) * 128]
        mask = tl > 0
        cnt = cnt + jnp.where(mask, 1.0, 0.0).astype(jnp.float32)
        for c in range(8):
            a = lp_ref[:, kk * 8 + c, :]
            bb = lt_ref[:, kk * 8 + c, :]
            d = jnp.where(mask, a - bb, 0.0)
            absd = jnp.abs(d)
            acc = acc + jnp.where(absd < 1.0, (0.5 * d) * d, absd - 0.5)
    sl1_out[0] += acc
    cnt_out[0] += cnt


def kernel(loc_preds, loc_targets, cls_preds, cls_targets):
    lp = loc_preds.reshape(B, KT, 128, C).transpose(0, 1, 3, 2).reshape(B, KT * C, 128)
    lt = loc_targets.reshape(B, KT, 128, C).transpose(0, 1, 3, 2).reshape(B, KT * C, 128)
    ct_sc = cls_targets.astype(jnp.int32).reshape(B // 8, 8, KT, 128).transpose(0, 2, 1, 3)
    ct_tc = cls_targets.astype(jnp.int32)

    mesh = plsc.VectorSubcoreMesh(
        core_axis_name="c", subcore_axis_name="s",
        num_cores=NC, num_subcores=NS)
    out_sc = pl.kernel(
        _sc_body,
        out_type=jax.ShapeDtypeStruct((NW, 2 * L), jnp.float32),
        mesh=mesh,
        scratch_types=[
            pltpu.VMEM((ROWS, 128), jnp.float32),
            pltpu.VMEM((ROWS, 128), jnp.float32),
            pltpu.VMEM((CK, 128), jnp.int32),
            pltpu.VMEM((2 * L,), jnp.float32),
        ],
    )(lp, lt, ct_sc)

    sl1_tc, cnt_tc = pl.pallas_call(
        _tc_body,
        grid=(NG, KTC // KB),
        in_specs=[
            pl.BlockSpec((8, KB * C, 128), lambda g, k: (g, k, 0)),
            pl.BlockSpec((8, KB * C, 128), lambda g, k: (g, k, 0)),
            pl.BlockSpec((8, KB * 128), lambda g, k: (g, k)),
        ],
        out_specs=[
            pl.BlockSpec((1, 8, 128), lambda g, k: (g, 0, 0)),
            pl.BlockSpec((1, 8, 128), lambda g, k: (g, 0, 0)),
        ],
        out_shape=[
            jax.ShapeDtypeStruct((NG, 8, 128), jnp.float32),
            jax.ShapeDtypeStruct((NG, 8, 128), jnp.float32),
        ],
        compiler_params=pltpu.CompilerParams(
            dimension_semantics=("parallel", "arbitrary")),
    )(lp, lt, ct_tc)

    sl1_sum = jnp.sum(out_sc[:, :L]) + jnp.sum(sl1_tc)
    n = jnp.sum(out_sc[:, L:]) + jnp.sum(cnt_tc)
    return 0.2 * (sl1_sum / n)


# speedup vs baseline: 1.5018x; 1.1297x over previous
import jax
import jax.numpy as jnp
from jax import lax
from jax.experimental import pallas as pl
from jax.experimental.pallas import tpu as pltpu, tpu_sc as plsc

NC, NS, L = 2, 16, 16
NW = NC * NS
B, A, C = 32, 65536, 8
KT = A // 128

KTC = 384
NG = B // 8
KB = 32
KPW = KT - KTC
CK = 32
NCHUNK = KPW // CK
ROWS = CK * C


def _sc_body(lp_hbm, lt_hbm, ct_hbm, out_hbm, lp_buf, lt_buf, ct_buf, res_buf):
    w = lax.axis_index("s") * NC + lax.axis_index("c")
    b = w
    r = b // 8
    i = b % 8
    kbase = KTC

    def chunk_body(c0, carry):
        k0 = kbase + c0 * CK
        pltpu.sync_copy(lp_hbm.at[b, pl.ds(k0 * C, ROWS), :], lp_buf)
        pltpu.sync_copy(lt_hbm.at[b, pl.ds(k0 * C, ROWS), :], lt_buf)
        pltpu.sync_copy(ct_hbm.at[r, pl.ds(k0, CK), i, :], ct_buf)

        def tile_body(kk, carry):
            accq, acca, acct, cnt = carry
            m = []
            for l in range(8):
                tl = ct_buf[kk, pl.ds(l * L, L)]
                ml = jnp.where(tl > 0, 1.0, 0.0).astype(jnp.float32)
                cnt = cnt + ml
                m.append(ml)
            for c in range(8):
                row = kk * 8 + c
                for l in range(8):
                    a = lp_buf[row, pl.ds(l * L, L)]
                    bb = lt_buf[row, pl.ds(l * L, L)]
                    d = (a - bb) * m[l]
                    absd = jnp.abs(d)
                    t = jnp.minimum(absd, 1.0)
                    accq = accq + (0.5 * t) * t
                    acca = acca + absd
                    acct = acct + t
            return accq, acca, acct, cnt

        return lax.fori_loop(0, CK, tile_body, carry)

    z = jnp.zeros((L,), jnp.float32)
    accq, acca, acct, cnt = lax.fori_loop(0, NCHUNK, chunk_body, (z, z, z, z))
    res_buf[pl.ds(0, L)] = accq + acca - acct
    res_buf[pl.ds(L, L)] = cnt
    pltpu.sync_copy(res_buf, out_hbm.at[w])


def _tc_body(lp_ref, lt_ref, ct_ref, sl1_out, cnt_out):
    kidx = pl.program_id(1)

    @pl.when(kidx == 0)
    def _():
        sl1_out[...] = jnp.zeros_like(sl1_out)
        cnt_out[...] = jnp.zeros_like(cnt_out)

    acc = jnp.zeros((8, 128), jnp.float32)
    cnt = jnp.zeros((8, 128), jnp.float32)
    for kk in range(KB):
        tl = ct_ref[:, kk * 128:(kk + 1) * 128]
        mask = tl > 0
        cnt = cnt + jnp.where(mask, 1.0, 0.0).astype(jnp.float32)
        for c in range(8):
            a = lp_ref[:, kk * 8 + c, :]
            bb = lt_ref[:, kk * 8 + c, :]
            d = jnp.where(mask, a - bb, 0.0)
            absd = jnp.abs(d)
            acc = acc + jnp.where(absd < 1.0, (0.5 * d) * d, absd - 0.5)
    sl1_out[0] += acc
    cnt_out[0] += cnt


def kernel(loc_preds, loc_targets, cls_preds, cls_targets):
    lp = loc_preds.reshape(B, KT, 128, C).transpose(0, 1, 3, 2).reshape(B, KT * C, 128)
    lt = loc_targets.reshape(B, KT, 128, C).transpose(0, 1, 3, 2).reshape(B, KT * C, 128)
    ct_sc = cls_targets.astype(jnp.int32).reshape(B // 8, 8, KT, 128).transpose(0, 2, 1, 3)
    ct_tc = cls_targets.astype(jnp.int32)

    mesh = plsc.VectorSubcoreMesh(
        core_axis_name="c", subcore_axis_name="s",
        num_cores=NC, num_subcores=NS)
    out_sc = pl.kernel(
        _sc_body,
        out_type=jax.ShapeDtypeStruct((NW, 2 * L), jnp.float32),
        mesh=mesh,
        scratch_types=[
            pltpu.VMEM((ROWS, 128), jnp.float32),
            pltpu.VMEM((ROWS, 128), jnp.float32),
            pltpu.VMEM((CK, 128), jnp.int32),
            pltpu.VMEM((2 * L,), jnp.float32),
        ],
    )(lp, lt, ct_sc)

    sl1_tc, cnt_tc = pl.pallas_call(
        _tc_body,
        grid=(NG, KTC // KB),
        in_specs=[
            pl.BlockSpec((8, KB * C, 128), lambda g, k: (g, k, 0)),
            pl.BlockSpec((8, KB * C, 128), lambda g, k: (g, k, 0)),
            pl.BlockSpec((8, KB * 128), lambda g, k: (g, k)),
        ],
        out_specs=[
            pl.BlockSpec((1, 8, 128), lambda g, k: (g, 0, 0)),
            pl.BlockSpec((1, 8, 128), lambda g, k: (g, 0, 0)),
        ],
        out_shape=[
            jax.ShapeDtypeStruct((NG, 8, 128), jnp.float32),
            jax.ShapeDtypeStruct((NG, 8, 128), jnp.float32),
        ],
        compiler_params=pltpu.CompilerParams(
            dimension_semantics=("parallel", "arbitrary")),
    )(lp, lt, ct_tc)

    sl1_sum = jnp.sum(out_sc[:, :L]) + jnp.sum(sl1_tc)
    n = jnp.sum(out_sc[:, L:]) + jnp.sum(cnt_tc)
    return 0.2 * (sl1_sum / n)
